# baseline (speedup 1.0000x reference)
import jax
import jax.numpy as jnp
from jax.experimental import pallas as pl
from jax.experimental.pallas import tpu as pltpu

F = 128
SELECT_NUM = 5
LP_LENGTH = 8
EMB_D = 768
BLK = 1024


N_COPY_CHUNKS = 32


def _msp_body(x_ref, k_ref, a_ref, p_ref, xb_ref,
              ek_ref, ev_ref, xbout_ref, copy_sem):
    i = pl.program_id(0)
    n = pl.num_programs(0)
    rows = xb_ref.shape[0] // N_COPY_CHUNKS

    @pl.when(i == 0)
    def _start_copy():
        for c in range(N_COPY_CHUNKS):
            pltpu.make_async_copy(
                xb_ref.at[pl.ds(c * rows, rows)],
                xbout_ref.at[pl.ds(c * rows, rows)],
                copy_sem).start()

    _op_body(x_ref, k_ref, a_ref, p_ref, ek_ref, ev_ref)

    @pl.when(i == n - 1)
    def _wait_copy():
        for c in range(N_COPY_CHUNKS):
            pltpu.make_async_copy(
                xb_ref.at[pl.ds(c * rows, rows)],
                xbout_ref.at[pl.ds(c * rows, rows)],
                copy_sem).wait()


def _op_body(x_ref, k_ref, a_ref, p_ref, ek_ref, ev_ref):
    x = x_ref[...]
    K = k_ref[...]
    A = a_ref[...]

    k_norm = jnp.sqrt(jnp.sum(K * K, axis=1, keepdims=True))
    nK = K / jnp.maximum(k_norm, 1e-12)

    dn = (((1,), (1,)), ((), ()))
    num = jax.lax.dot_general(x, A * nK, dn,
                              preferred_element_type=jnp.float32,
                              precision=jax.lax.Precision.HIGHEST)
    den2 = jax.lax.dot_general(x * x, A * A, dn,
                               preferred_element_type=jnp.float32,
                               precision=jax.lax.Precision.HIGHEST)
    den = jnp.maximum(jnp.sqrt(den2), 1e-12)
    scores = num / den

    iota = jax.lax.broadcasted_iota(jnp.int32, (BLK, F), 1)
    cur = scores
    w = jnp.zeros_like(scores)
    for _ in range(SELECT_NUM):
        mx = jnp.max(cur, axis=1, keepdims=True)
        elig = cur == mx
        first = jnp.min(jnp.where(elig, iota, F), axis=1, keepdims=True)
        sel = iota == first
        w = jnp.where(sel, scores, w)
        cur = jnp.where(sel, -jnp.inf, cur)

    p = p_ref[...]
    half = p.shape[1] // 2
    ek_ref[...] = jnp.dot(w, p[:, :half], preferred_element_type=jnp.float32)
    ev_ref[...] = jnp.dot(w, p[:, half:], preferred_element_type=jnp.float32)


@jax.jit
def _msp(x_querry, p_flat, lk, la, x_block):
    B = x_querry.shape[0]
    D = p_flat.shape[1]
    half = D // 2
    ek, ev, xb_out = pl.pallas_call(
        _msp_body,
        grid=(B // BLK,),
        in_specs=[
            pl.BlockSpec((BLK, x_querry.shape[1]), lambda i: (i, 0)),
            pl.BlockSpec((F, lk.shape[1]), lambda i: (0, 0)),
            pl.BlockSpec((F, la.shape[1]), lambda i: (0, 0)),
            pl.BlockSpec((F, D), lambda i: (0, 0)),
            pl.BlockSpec(memory_space=pl.ANY),
        ],
        out_specs=[
            pl.BlockSpec((BLK, half), lambda i: (i, 0)),
            pl.BlockSpec((BLK, half), lambda i: (i, 0)),
            pl.BlockSpec(memory_space=pl.ANY),
        ],
        out_shape=[
            jax.ShapeDtypeStruct((B, half), jnp.float32),
            jax.ShapeDtypeStruct((B, half), jnp.float32),
            jax.ShapeDtypeStruct(x_block.shape, x_block.dtype),
        ],
        scratch_shapes=[pltpu.SemaphoreType.DMA],
    )(x_querry, lk, la, p_flat, x_block)
    return ek, ev, xb_out


def kernel(x_querry, l, x_block, lp, lk, la):
    B = x_querry.shape[0]
    p_flat = lp[:F].reshape(F, LP_LENGTH * EMB_D)
    ek, ev, xb_out = _msp(x_querry, p_flat, lk, la, x_block)
    i = LP_LENGTH // 2
    return (ek.reshape(B, i, EMB_D), ev.reshape(B, i, EMB_D),
            jnp.float32(0.0), xb_out)


# speedup vs baseline: 45.6766x; 45.6766x over previous
import jax
import jax.numpy as jnp
from jax.experimental import pallas as pl
from jax.experimental.pallas import tpu as pltpu

F = 128
SELECT_NUM = 5
LP_LENGTH = 8
EMB_D = 768
BLK = 512


def _msp_body(x_ref, k_ref, a_ref, p_ref, ek_ref, ev_ref):
    x = x_ref[...]
    K = k_ref[...]
    A = a_ref[...]

    k_norm = jnp.sqrt(jnp.sum(K * K, axis=1, keepdims=True))
    nK = K / jnp.maximum(k_norm, 1e-12)

    dn = (((1,), (1,)), ((), ()))
    num = jax.lax.dot_general(x, A * nK, dn,
                              preferred_element_type=jnp.float32,
                              precision=jax.lax.Precision.HIGHEST)
    den2 = jax.lax.dot_general(x * x, A * A, dn,
                               preferred_element_type=jnp.float32,
                               precision=jax.lax.Precision.HIGHEST)
    den = jnp.maximum(jnp.sqrt(den2), 1e-12)
    scores = num / den

    iota = jax.lax.broadcasted_iota(jnp.int32, (BLK, F), 1)
    cur = scores
    w = jnp.zeros_like(scores)
    for _ in range(SELECT_NUM):
        mx = jnp.max(cur, axis=1, keepdims=True)
        elig = cur == mx
        first = jnp.min(jnp.where(elig, iota, F), axis=1, keepdims=True)
        sel = iota == first
        w = jnp.where(sel, scores, w)
        cur = jnp.where(sel, -jnp.inf, cur)

    p = p_ref[...]
    half = p.shape[1] // 2
    ek_ref[...] = jnp.dot(w, p[:, :half], preferred_element_type=jnp.float32)
    ev_ref[...] = jnp.dot(w, p[:, half:], preferred_element_type=jnp.float32)


@jax.jit
def _msp(x_querry, p_flat, lk, la):
    B = x_querry.shape[0]
    D = p_flat.shape[1]
    half = D // 2
    ek, ev = pl.pallas_call(
        _msp_body,
        grid=(B // BLK,),
        in_specs=[
            pl.BlockSpec((BLK, x_querry.shape[1]), lambda i: (i, 0)),
            pl.BlockSpec((F, lk.shape[1]), lambda i: (0, 0)),
            pl.BlockSpec((F, la.shape[1]), lambda i: (0, 0)),
            pl.BlockSpec((F, D), lambda i: (0, 0)),
        ],
        out_specs=[
            pl.BlockSpec((BLK, half), lambda i: (i, 0)),
            pl.BlockSpec((BLK, half), lambda i: (i, 0)),
        ],
        out_shape=[
            jax.ShapeDtypeStruct((B, half), jnp.float32),
            jax.ShapeDtypeStruct((B, half), jnp.float32),
        ],
    )(x_querry, lk, la, p_flat)
    return ek, ev


def kernel(x_querry, l, x_block, lp, lk, la):
    B = x_querry.shape[0]
    p_flat = lp[:F].reshape(F, LP_LENGTH * EMB_D)
    ek, ev = _msp(x_querry, p_flat, lk, la)
    i = LP_LENGTH // 2
    return (ek.reshape(B, i, EMB_D), ev.reshape(B, i, EMB_D),
            jnp.float32(0.0), x_block)
